# baseline (speedup 1.0000x reference)
import functools

import jax
import jax.numpy as jnp
from jax import lax
from jax.experimental import pallas as pl
from jax.experimental.pallas import tpu as pltpu
from jax.experimental.pallas import tpu_sc as plsc

T = 2048
D = 1024
E = 16
F = 512
SF = 1024
SCALE = 2.5
GSIZE = 4
NGROUP = 4
BT = 256
NSLOT = 8192
NT = NSLOT // BT
NEG = -1e30
WL = 128

NJ = 64

NW = 32
CH = T // NW
CQ = CH // 4


def _routing_body(x_ref, gw_ref, bias_ref, slot1_ref, slot2_ref,
                  w1_ref, w2_ref, te_ref, tmb_ref, valid_ref, p_ref):
    x = x_ref[...]
    logits = jnp.dot(x, gw_ref[...], preferred_element_type=jnp.float32)
    scores = jax.nn.sigmoid(logits)
    sb = scores + bias_ref[...]
    col = lax.broadcasted_iota(jnp.int32, (T, E), 1)

    gscores = []
    for g in range(NGROUP):
        ing = (col // GSIZE) == g
        vg = jnp.where(ing, sb, NEG)
        m1 = jnp.max(vg, axis=1, keepdims=True)
        j1 = jnp.min(jnp.where(vg == m1, col, E), axis=1, keepdims=True)
        m2 = jnp.max(jnp.where(col == j1, NEG, vg), axis=1, keepdims=True)
        gscores.append(m1 + m2)
    G = jnp.concatenate(gscores, axis=1)
    col4 = lax.broadcasted_iota(jnp.int32, (T, NGROUP), 1)
    gm1 = jnp.max(G, axis=1, keepdims=True)
    g1 = jnp.min(jnp.where(G == gm1, col4, NGROUP), axis=1, keepdims=True)
    G2 = jnp.where(col4 == g1, NEG, G)
    gm2 = jnp.max(G2, axis=1, keepdims=True)
    g2 = jnp.min(jnp.where(G2 == gm2, col4, NGROUP), axis=1, keepdims=True)

    keep = ((col // GSIZE) == g1) | ((col // GSIZE) == g2)
    masked = jnp.where(keep, sb, NEG)
    m1 = jnp.max(masked, axis=1, keepdims=True)
    e1 = jnp.min(jnp.where(masked == m1, col, E), axis=1, keepdims=True)
    m2v = jnp.where(col == e1, NEG, masked)
    m2 = jnp.max(m2v, axis=1, keepdims=True)
    e2 = jnp.min(jnp.where(m2v == m2, col, E), axis=1, keepdims=True)

    w1 = jnp.sum(jnp.where(col == e1, scores, 0.0), axis=1, keepdims=True)
    w2 = jnp.sum(jnp.where(col == e2, scores, 0.0), axis=1, keepdims=True)
    ws = w1 + w2 + 1e-20
    w1_ref[...] = jnp.broadcast_to(w1 / ws * SCALE, (T, WL))
    w2_ref[...] = jnp.broadcast_to(w2 / ws * SCALE, (T, WL))

    oh1 = (col == e1).astype(jnp.float32)
    oh2 = (col == e2).astype(jnp.float32)
    cnt = oh1 + oh2
    tri = (lax.broadcasted_iota(jnp.int32, (BT, BT), 0)
           > lax.broadcasted_iota(jnp.int32, (BT, BT), 1)).astype(jnp.float32)
    carry = jnp.zeros((1, E), jnp.float32)
    for i in range(T // BT):
        c = cnt[i * BT:(i + 1) * BT, :]
        p_ref[i * BT:(i + 1) * BT, :] = (
            jnp.dot(tri, c, preferred_element_type=jnp.float32) + carry)
        carry = carry + jnp.sum(c, axis=0, keepdims=True)

    n_e = jnp.ceil(carry * (1.0 / BT))
    s_e = n_e * BT
    tri16 = (lax.broadcasted_iota(jnp.int32, (E, E), 0)
             < lax.broadcasted_iota(jnp.int32, (E, E), 1)).astype(jnp.float32)
    offs = jnp.dot(s_e, tri16, preferred_element_type=jnp.float32)

    slotf = offs + p_ref[...]
    slot1_ref[...] = jnp.sum(jnp.where(col == e1, slotf, 0.0), axis=1,
                             keepdims=True).astype(jnp.int32)
    slot2_ref[...] = jnp.sum(jnp.where(col == e2, slotf, 0.0), axis=1,
                             keepdims=True).astype(jnp.int32)
    first_blk = offs * (1.0 / BT)
    start_e = jnp.dot(n_e, tri16, preferred_element_type=jnp.float32)
    total = jnp.sum(n_e)
    jrow = lax.broadcasted_iota(jnp.int32, (NJ, E), 0).astype(jnp.float32)
    colj = lax.broadcasted_iota(jnp.int32, (NJ, E), 1)
    startb = jnp.broadcast_to(start_e, (NJ, E))
    fblkb = jnp.broadcast_to(first_blk, (NJ, E))
    te = jnp.sum((jrow >= startb).astype(jnp.float32), axis=1,
                 keepdims=True) - 1.0
    tei = te.astype(jnp.int32)
    tstart = jnp.sum(jnp.where(colj == tei, startb, 0.0), axis=1, keepdims=True)
    tfirst = jnp.sum(jnp.where(colj == tei, fblkb, 0.0), axis=1, keepdims=True)
    jcol = jrow[:, 0:1]
    tmb = tfirst + (jcol - tstart)
    valid = jcol < total
    te_last = jnp.max(jnp.where(valid, te, -1.0))
    tmb_last = jnp.max(jnp.where(valid, tmb, -1.0))
    te_ref[...] = jnp.where(valid, te, te_last).astype(jnp.int32)
    tmb_ref[...] = jnp.where(valid, tmb, tmb_last).astype(jnp.int32)
    valid_ref[...] = valid.astype(jnp.int32)


def _routing_call(x, gate_w, bias2):
    return pl.pallas_call(
        _routing_body,
        out_shape=[
            jax.ShapeDtypeStruct((T, 1), jnp.int32),
            jax.ShapeDtypeStruct((T, 1), jnp.int32),
            jax.ShapeDtypeStruct((T, WL), jnp.float32),
            jax.ShapeDtypeStruct((T, WL), jnp.float32),
            jax.ShapeDtypeStruct((NJ, 1), jnp.int32),
            jax.ShapeDtypeStruct((NJ, 1), jnp.int32),
            jax.ShapeDtypeStruct((NJ, 1), jnp.int32),
        ],
        scratch_shapes=[pltpu.VMEM((T, E), jnp.float32)],
    )(x, gate_w, bias2)


def _dispatch_body(x_hbm, s1_hbm, s2_hbm, w1_hbm, w2_hbm, xs_hbm, cwt_hbm,
                   xrows, s1v, s2v, w1v, w2v, sem1, sem2):
    wid = lax.axis_index("s") * 2 + lax.axis_index("c")
    base = wid * CH
    pltpu.sync_copy(x_hbm.at[pl.ds(base, CH)], xrows)
    pltpu.sync_copy(s1_hbm.at[pl.ds(base, CH)], s1v)
    pltpu.sync_copy(s2_hbm.at[pl.ds(base, CH)], s2v)
    pltpu.sync_copy(w1_hbm.at[pl.ds(base, CH)], w1v)
    pltpu.sync_copy(w2_hbm.at[pl.ds(base, CH)], w2v)
    c1 = pltpu.async_copy(xrows, xs_hbm.at[s1v], sem1)
    c2 = pltpu.async_copy(xrows, xs_hbm.at[s2v], sem1)
    c3 = pltpu.async_copy(w1v, cwt_hbm.at[s1v], sem2)
    c4 = pltpu.async_copy(w2v, cwt_hbm.at[s2v], sem2)
    c1.wait()
    c2.wait()
    c3.wait()
    c4.wait()


def _ffn_body(te_ref, tmb_ref, valid_ref, xs_ref, egu_ref, edown_ref,
              cwt_ref, x_ref, sgu_ref, sdown_ref, ys_ref, shared_ref):
    i = pl.program_id(0)

    @pl.when(valid_ref[i] == 1)
    def _():
        xb = xs_ref[...].astype(jnp.bfloat16)
        gu = jnp.dot(xb, egu_ref[0].astype(jnp.bfloat16),
                     preferred_element_type=jnp.float32)
        a = gu[:, :F] * jax.nn.sigmoid(gu[:, :F]) * gu[:, F:]
        y = jnp.dot(a.astype(jnp.bfloat16), edown_ref[0].astype(jnp.bfloat16),
                    preferred_element_type=jnp.float32)
        ys_ref[...] = y * cwt_ref[:, 0:1]

    @pl.when(i < T // BD)
    def _():
        g = jnp.dot(x_ref[...].astype(jnp.bfloat16),
                    sgu_ref[...].astype(jnp.bfloat16),
                    preferred_element_type=jnp.float32)
        sa = g[:, :SF] * jax.nn.sigmoid(g[:, :SF]) * g[:, SF:]
        shared_ref[...] = jnp.dot(sa.astype(jnp.bfloat16),
                                  sdown_ref[...].astype(jnp.bfloat16),
                                  preferred_element_type=jnp.float32)


def _ffn_call(te, tmb, valid, xs, egu, edown, cwt, x, sgu, sdown):
    nd = T // BD
    grid_spec = pltpu.PrefetchScalarGridSpec(
        num_scalar_prefetch=3,
        grid=(NT,),
        in_specs=[
            pl.BlockSpec((BT, D), lambda i, te, tmb, valid: (tmb[i], 0)),
            pl.BlockSpec((1, D, 2 * F), lambda i, te, tmb, valid: (te[i], 0, 0)),
            pl.BlockSpec((1, F, D), lambda i, te, tmb, valid: (te[i], 0, 0)),
            pl.BlockSpec((BT, WL), lambda i, te, tmb, valid: (tmb[i], 0)),
            pl.BlockSpec((BD, D),
                         lambda i, te, tmb, valid: (jnp.minimum(i, nd - 1), 0)),
            pl.BlockSpec((D, 2 * SF), lambda i, te, tmb, valid: (0, 0)),
            pl.BlockSpec((SF, D), lambda i, te, tmb, valid: (0, 0)),
        ],
        out_specs=[
            pl.BlockSpec((BT, D), lambda i, te, tmb, valid: (tmb[i], 0)),
            pl.BlockSpec((BD, D),
                         lambda i, te, tmb, valid: (jnp.minimum(i, nd - 1), 0)),
        ],
    )
    return pl.pallas_call(
        _ffn_body,
        grid_spec=grid_spec,
        out_shape=[
            jax.ShapeDtypeStruct((NSLOT, D), jnp.float32),
            jax.ShapeDtypeStruct((T, D), jnp.float32),
        ],
    )(te, tmb, valid, xs, egu, edown, cwt, x, sgu, sdown)


def _collect_body(ys_hbm, s1_hbm, s2_hbm, shared_hbm, out_hbm,
                  buf1, buf2, buf3, buf4, buf5, buf6,
                  s1v, s2v, s3v, s4v, semA, semB, semS0, semS1):
    wid = lax.axis_index("s") * 2 + lax.axis_index("c")
    sets = [(buf1, buf2, buf3, s1v, s2v, semA, semS0),
            (buf4, buf5, buf6, s3v, s4v, semB, semS1)]
    NQ = CH // CQ

    def _gathers(q, st):
        b1, b2, b3, sv1, sv2, gsem, _ = st
        base = wid * CH + q * CQ
        pltpu.sync_copy(s1_hbm.at[pl.ds(base, CQ)], sv1)
        pltpu.sync_copy(s2_hbm.at[pl.ds(base, CQ)], sv2)
        return (pltpu.async_copy(ys_hbm.at[sv1], b1, gsem),
                pltpu.async_copy(ys_hbm.at[sv2], b2, gsem),
                pltpu.async_copy(shared_hbm.at[pl.ds(base, CQ)], b3, gsem))

    def _add(st):
        b1, b2, b3 = st[0], st[1], st[2]

        def _row(r, _):
            for jj in range(D // 16):
                sl = pl.ds(jj * 16, 16)
                b1[r, sl] = b1[r, sl] + b2[r, sl] + b3[r, sl]
            return _

        lax.fori_loop(0, CQ, _row, 0)

    descs = {0: _gathers(0, sets[0])}
    stores = [None, None]
    for q in range(NQ):
        s = q % 2
        nxt = q + 1
        if nxt < NQ:
            if stores[nxt % 2] is not None:
                stores[nxt % 2].wait()
                stores[nxt % 2] = None
            descs[nxt] = _gathers(nxt, sets[nxt % 2])
        for g in descs[q]:
            g.wait()
        _add(sets[s])
        stores[s] = pltpu.async_copy(
            sets[s][0], out_hbm.at[pl.ds(wid * CH + q * CQ, CQ)], sets[s][6])
    for st in stores:
        if st is not None:
            st.wait()


@functools.lru_cache(maxsize=1)
def _sc_kernels():
    mesh = plsc.VectorSubcoreMesh(core_axis_name="c", subcore_axis_name="s")
    dispatch = pl.kernel(
        _dispatch_body,
        mesh=mesh,
        out_type=[
            jax.ShapeDtypeStruct((NSLOT, D), jnp.float32),
            jax.ShapeDtypeStruct((NSLOT, WL), jnp.float32),
        ],
        scratch_types=[
            pltpu.VMEM((CH, D), jnp.float32),
            pltpu.VMEM((CH,), jnp.int32),
            pltpu.VMEM((CH,), jnp.int32),
            pltpu.VMEM((CH, WL), jnp.float32),
            pltpu.VMEM((CH, WL), jnp.float32),
            pltpu.SemaphoreType.DMA,
            pltpu.SemaphoreType.DMA,
        ],
    )
    collect = pl.kernel(
        _collect_body,
        mesh=mesh,
        out_type=jax.ShapeDtypeStruct((T, D), jnp.float32),
        scratch_types=[
            pltpu.VMEM((CQ, D), jnp.float32),
            pltpu.VMEM((CQ, D), jnp.float32),
            pltpu.VMEM((CQ, D), jnp.float32),
            pltpu.VMEM((CQ, D), jnp.float32),
            pltpu.VMEM((CQ, D), jnp.float32),
            pltpu.VMEM((CQ, D), jnp.float32),
            pltpu.VMEM((CQ,), jnp.int32),
            pltpu.VMEM((CQ,), jnp.int32),
            pltpu.VMEM((CQ,), jnp.int32),
            pltpu.VMEM((CQ,), jnp.int32),
            pltpu.SemaphoreType.DMA,
            pltpu.SemaphoreType.DMA,
            pltpu.SemaphoreType.DMA,
            pltpu.SemaphoreType.DMA,
        ],
    )
    return dispatch, collect


def _dispatch(x, s1, s2, w1x, w2x):
    return _sc_kernels()[0](x, s1, s2, w1x, w2x)


def _collect(ys, s1, s2, shared):
    return _sc_kernels()[1](ys, s1, s2, shared)


BD = 256


def _shared_body(x_ref, sgu_ref, sdown_ref, out_ref):
    g = jnp.dot(x_ref[...].astype(jnp.bfloat16), sgu_ref[...].astype(jnp.bfloat16),
                preferred_element_type=jnp.float32)
    a = g[:, :SF] * jax.nn.sigmoid(g[:, :SF]) * g[:, SF:]
    out_ref[...] = jnp.dot(a.astype(jnp.bfloat16), sdown_ref[...].astype(jnp.bfloat16),
                           preferred_element_type=jnp.float32)


def _shared_call(x, sgu, sdown):
    return pl.pallas_call(
        _shared_body,
        grid=(T // BD,),
        in_specs=[
            pl.BlockSpec((BD, D), lambda i: (i, 0)),
            pl.BlockSpec((D, 2 * SF), lambda i: (0, 0)),
            pl.BlockSpec((SF, D), lambda i: (0, 0)),
        ],
        out_specs=pl.BlockSpec((BD, D), lambda i: (i, 0)),
        out_shape=jax.ShapeDtypeStruct((T, D), jnp.float32),
    )(x, sgu, sdown)


def kernel(hidden_states, gate_w, e_score_correction_bias, expert_gate_up,
           expert_down, shared_gate_up, shared_down):
    x = hidden_states
    bias2 = e_score_correction_bias.reshape(1, E)
    slot1, slot2, w1x, w2x, te, tmb, valid = _routing_call(x, gate_w, bias2)
    s1 = slot1.reshape(T)
    s2 = slot2.reshape(T)
    te = te.reshape(NJ)
    tmb = tmb.reshape(NJ)
    valid = valid.reshape(NJ)

    xs, cwt = _dispatch(x, s1, s2, w1x, w2x)
    ys, shared = _ffn_call(te, tmb, valid, xs, expert_gate_up, expert_down,
                           cwt, x, shared_gate_up, shared_down)
    return _collect(ys, s1, s2, shared)


# speedup vs baseline: 1.0082x; 1.0082x over previous
import functools

import jax
import jax.numpy as jnp
from jax import lax
from jax.experimental import pallas as pl
from jax.experimental.pallas import tpu as pltpu
from jax.experimental.pallas import tpu_sc as plsc

T = 2048
D = 1024
E = 16
F = 512
SF = 1024
SCALE = 2.5
GSIZE = 4
NGROUP = 4
BT = 256
NSLOT = 8192
NT = NSLOT // BT
NEG = -1e30
WL = 128

NJ = 64

NW = 32
CH = T // NW
CQ = CH // 4


def _routing_body(x_ref, gw_ref, bias_ref, slot1_ref, slot2_ref,
                  w1_ref, w2_ref, te_ref, tmb_ref, valid_ref, p_ref):
    x = x_ref[...]
    logits = jnp.dot(x, gw_ref[...], preferred_element_type=jnp.float32)
    scores = jax.nn.sigmoid(logits)
    sb = scores + bias_ref[...]
    col = lax.broadcasted_iota(jnp.int32, (T, E), 1)

    gscores = []
    for g in range(NGROUP):
        ing = (col // GSIZE) == g
        vg = jnp.where(ing, sb, NEG)
        m1 = jnp.max(vg, axis=1, keepdims=True)
        j1 = jnp.min(jnp.where(vg == m1, col, E), axis=1, keepdims=True)
        m2 = jnp.max(jnp.where(col == j1, NEG, vg), axis=1, keepdims=True)
        gscores.append(m1 + m2)
    G = jnp.concatenate(gscores, axis=1)
    col4 = lax.broadcasted_iota(jnp.int32, (T, NGROUP), 1)
    gm1 = jnp.max(G, axis=1, keepdims=True)
    g1 = jnp.min(jnp.where(G == gm1, col4, NGROUP), axis=1, keepdims=True)
    G2 = jnp.where(col4 == g1, NEG, G)
    gm2 = jnp.max(G2, axis=1, keepdims=True)
    g2 = jnp.min(jnp.where(G2 == gm2, col4, NGROUP), axis=1, keepdims=True)

    keep = ((col // GSIZE) == g1) | ((col // GSIZE) == g2)
    masked = jnp.where(keep, sb, NEG)
    m1 = jnp.max(masked, axis=1, keepdims=True)
    e1 = jnp.min(jnp.where(masked == m1, col, E), axis=1, keepdims=True)
    m2v = jnp.where(col == e1, NEG, masked)
    m2 = jnp.max(m2v, axis=1, keepdims=True)
    e2 = jnp.min(jnp.where(m2v == m2, col, E), axis=1, keepdims=True)

    w1 = jnp.sum(jnp.where(col == e1, scores, 0.0), axis=1, keepdims=True)
    w2 = jnp.sum(jnp.where(col == e2, scores, 0.0), axis=1, keepdims=True)
    ws = w1 + w2 + 1e-20
    w1_ref[...] = jnp.broadcast_to(w1 / ws * SCALE, (T, WL))
    w2_ref[...] = jnp.broadcast_to(w2 / ws * SCALE, (T, WL))

    oh1 = (col == e1).astype(jnp.float32)
    oh2 = (col == e2).astype(jnp.float32)
    cnt = oh1 + oh2
    tri = (lax.broadcasted_iota(jnp.int32, (BT, BT), 0)
           > lax.broadcasted_iota(jnp.int32, (BT, BT), 1)).astype(jnp.float32)
    carry = jnp.zeros((1, E), jnp.float32)
    for i in range(T // BT):
        c = cnt[i * BT:(i + 1) * BT, :]
        p_ref[i * BT:(i + 1) * BT, :] = (
            jnp.dot(tri, c, preferred_element_type=jnp.float32) + carry)
        carry = carry + jnp.sum(c, axis=0, keepdims=True)

    n_e = jnp.ceil(carry * (1.0 / BT))
    s_e = n_e * BT
    tri16 = (lax.broadcasted_iota(jnp.int32, (E, E), 0)
             < lax.broadcasted_iota(jnp.int32, (E, E), 1)).astype(jnp.float32)
    offs = jnp.dot(s_e, tri16, preferred_element_type=jnp.float32)

    slotf = offs + p_ref[...]
    slot1_ref[...] = jnp.sum(jnp.where(col == e1, slotf, 0.0), axis=1,
                             keepdims=True).astype(jnp.int32)
    slot2_ref[...] = jnp.sum(jnp.where(col == e2, slotf, 0.0), axis=1,
                             keepdims=True).astype(jnp.int32)
    first_blk = offs * (1.0 / BT)
    start_e = jnp.dot(n_e, tri16, preferred_element_type=jnp.float32)
    total = jnp.sum(n_e)
    jrow = lax.broadcasted_iota(jnp.int32, (NJ, E), 0).astype(jnp.float32)
    colj = lax.broadcasted_iota(jnp.int32, (NJ, E), 1)
    startb = jnp.broadcast_to(start_e, (NJ, E))
    fblkb = jnp.broadcast_to(first_blk, (NJ, E))
    te = jnp.sum((jrow >= startb).astype(jnp.float32), axis=1,
                 keepdims=True) - 1.0
    tei = te.astype(jnp.int32)
    tstart = jnp.sum(jnp.where(colj == tei, startb, 0.0), axis=1, keepdims=True)
    tfirst = jnp.sum(jnp.where(colj == tei, fblkb, 0.0), axis=1, keepdims=True)
    jcol = jrow[:, 0:1]
    tmb = tfirst + (jcol - tstart)
    valid = jcol < total
    te_last = jnp.max(jnp.where(valid, te, -1.0))
    tmb_last = jnp.max(jnp.where(valid, tmb, -1.0))
    te_ref[...] = jnp.where(valid, te, te_last).astype(jnp.int32)
    tmb_ref[...] = jnp.where(valid, tmb, tmb_last).astype(jnp.int32)
    valid_ref[...] = valid.astype(jnp.int32)


def _routing_call(x, gate_w, bias2):
    return pl.pallas_call(
        _routing_body,
        out_shape=[
            jax.ShapeDtypeStruct((T, 1), jnp.int32),
            jax.ShapeDtypeStruct((T, 1), jnp.int32),
            jax.ShapeDtypeStruct((T, WL), jnp.float32),
            jax.ShapeDtypeStruct((T, WL), jnp.float32),
            jax.ShapeDtypeStruct((NJ, 1), jnp.int32),
            jax.ShapeDtypeStruct((NJ, 1), jnp.int32),
            jax.ShapeDtypeStruct((NJ, 1), jnp.int32),
        ],
        scratch_shapes=[pltpu.VMEM((T, E), jnp.float32)],
    )(x, gate_w, bias2)


def _dispatch_body(x_hbm, s1_hbm, s2_hbm, w1_hbm, w2_hbm, xs_hbm, cwt_hbm,
                   xrows, s1v, s2v, w1v, w2v, sem1, sem2):
    wid = lax.axis_index("s") * 2 + lax.axis_index("c")
    base = wid * CH
    pltpu.sync_copy(x_hbm.at[pl.ds(base, CH)], xrows)
    pltpu.sync_copy(s1_hbm.at[pl.ds(base, CH)], s1v)
    pltpu.sync_copy(s2_hbm.at[pl.ds(base, CH)], s2v)
    pltpu.sync_copy(w1_hbm.at[pl.ds(base, CH)], w1v)
    pltpu.sync_copy(w2_hbm.at[pl.ds(base, CH)], w2v)
    c1 = pltpu.async_copy(xrows, xs_hbm.at[s1v], sem1)
    c2 = pltpu.async_copy(xrows, xs_hbm.at[s2v], sem1)
    c3 = pltpu.async_copy(w1v, cwt_hbm.at[s1v], sem2)
    c4 = pltpu.async_copy(w2v, cwt_hbm.at[s2v], sem2)
    c1.wait()
    c2.wait()
    c3.wait()
    c4.wait()


def _ffn_body(te_ref, tmb_ref, valid_ref, xs_ref, egu_ref, edown_ref,
              cwt_ref, ys_ref):
    i = pl.program_id(0)

    @pl.when(valid_ref[i] == 1)
    def _():
        xb = xs_ref[...].astype(jnp.bfloat16)
        gu = jnp.dot(xb, egu_ref[0].astype(jnp.bfloat16),
                     preferred_element_type=jnp.float32)
        a = gu[:, :F] * jax.nn.sigmoid(gu[:, :F]) * gu[:, F:]
        y = jnp.dot(a.astype(jnp.bfloat16), edown_ref[0].astype(jnp.bfloat16),
                    preferred_element_type=jnp.float32)
        ys_ref[...] = y * cwt_ref[:, 0:1]


def _ffn_call(te, tmb, valid, xs, egu, edown, cwt):
    grid_spec = pltpu.PrefetchScalarGridSpec(
        num_scalar_prefetch=3,
        grid=(NT,),
        in_specs=[
            pl.BlockSpec((BT, D), lambda i, te, tmb, valid: (tmb[i], 0)),
            pl.BlockSpec((1, D, 2 * F), lambda i, te, tmb, valid: (te[i], 0, 0)),
            pl.BlockSpec((1, F, D), lambda i, te, tmb, valid: (te[i], 0, 0)),
            pl.BlockSpec((BT, WL), lambda i, te, tmb, valid: (tmb[i], 0)),
        ],
        out_specs=pl.BlockSpec((BT, D), lambda i, te, tmb, valid: (tmb[i], 0)),
    )
    return pl.pallas_call(
        _ffn_body,
        grid_spec=grid_spec,
        out_shape=jax.ShapeDtypeStruct((NSLOT, D), jnp.float32),
    )(te, tmb, valid, xs, egu, edown, cwt)


def _collect_body(ys_hbm, s1_hbm, s2_hbm, shared_hbm, out_hbm,
                  buf1, buf2, buf3, buf4, buf5, buf6,
                  s1v, s2v, s3v, s4v, semA, semB, semS0, semS1):
    wid = lax.axis_index("s") * 2 + lax.axis_index("c")
    sets = [(buf1, buf2, buf3, s1v, s2v, semA, semS0),
            (buf4, buf5, buf6, s3v, s4v, semB, semS1)]
    NQ = CH // CQ

    def _gathers(q, st):
        b1, b2, b3, sv1, sv2, gsem, _ = st
        base = wid * CH + q * CQ
        pltpu.sync_copy(s1_hbm.at[pl.ds(base, CQ)], sv1)
        pltpu.sync_copy(s2_hbm.at[pl.ds(base, CQ)], sv2)
        return (pltpu.async_copy(ys_hbm.at[sv1], b1, gsem),
                pltpu.async_copy(ys_hbm.at[sv2], b2, gsem),
                pltpu.async_copy(shared_hbm.at[pl.ds(base, CQ)], b3, gsem))

    def _add(st):
        b1, b2, b3 = st[0], st[1], st[2]

        def _row(r, _):
            for jj in range(D // 16):
                sl = pl.ds(jj * 16, 16)
                b1[r, sl] = b1[r, sl] + b2[r, sl] + b3[r, sl]
            return _

        lax.fori_loop(0, CQ, _row, 0)

    descs = {0: _gathers(0, sets[0])}
    stores = [None, None]
    for q in range(NQ):
        s = q % 2
        nxt = q + 1
        if nxt < NQ:
            if stores[nxt % 2] is not None:
                stores[nxt % 2].wait()
                stores[nxt % 2] = None
            descs[nxt] = _gathers(nxt, sets[nxt % 2])
        for g in descs[q]:
            g.wait()
        _add(sets[s])
        stores[s] = pltpu.async_copy(
            sets[s][0], out_hbm.at[pl.ds(wid * CH + q * CQ, CQ)], sets[s][6])
    for st in stores:
        if st is not None:
            st.wait()


@functools.lru_cache(maxsize=1)
def _sc_kernels():
    mesh = plsc.VectorSubcoreMesh(core_axis_name="c", subcore_axis_name="s")
    dispatch = pl.kernel(
        _dispatch_body,
        mesh=mesh,
        out_type=[
            jax.ShapeDtypeStruct((NSLOT, D), jnp.float32),
            jax.ShapeDtypeStruct((NSLOT, WL), jnp.float32),
        ],
        scratch_types=[
            pltpu.VMEM((CH, D), jnp.float32),
            pltpu.VMEM((CH,), jnp.int32),
            pltpu.VMEM((CH,), jnp.int32),
            pltpu.VMEM((CH, WL), jnp.float32),
            pltpu.VMEM((CH, WL), jnp.float32),
            pltpu.SemaphoreType.DMA,
            pltpu.SemaphoreType.DMA,
        ],
    )
    collect = pl.kernel(
        _collect_body,
        mesh=mesh,
        out_type=jax.ShapeDtypeStruct((T, D), jnp.float32),
        scratch_types=[
            pltpu.VMEM((CQ, D), jnp.float32),
            pltpu.VMEM((CQ, D), jnp.float32),
            pltpu.VMEM((CQ, D), jnp.float32),
            pltpu.VMEM((CQ, D), jnp.float32),
            pltpu.VMEM((CQ, D), jnp.float32),
            pltpu.VMEM((CQ, D), jnp.float32),
            pltpu.VMEM((CQ,), jnp.int32),
            pltpu.VMEM((CQ,), jnp.int32),
            pltpu.VMEM((CQ,), jnp.int32),
            pltpu.VMEM((CQ,), jnp.int32),
            pltpu.SemaphoreType.DMA,
            pltpu.SemaphoreType.DMA,
            pltpu.SemaphoreType.DMA,
            pltpu.SemaphoreType.DMA,
        ],
    )
    return dispatch, collect


def _dispatch(x, s1, s2, w1x, w2x):
    return _sc_kernels()[0](x, s1, s2, w1x, w2x)


def _collect(ys, s1, s2, shared):
    return _sc_kernels()[1](ys, s1, s2, shared)


BD = 256


def _shared_body(x_ref, sgu_ref, sdown_ref, out_ref):
    g = jnp.dot(x_ref[...].astype(jnp.bfloat16), sgu_ref[...].astype(jnp.bfloat16),
                preferred_element_type=jnp.float32)
    a = g[:, :SF] * jax.nn.sigmoid(g[:, :SF]) * g[:, SF:]
    out_ref[...] = jnp.dot(a.astype(jnp.bfloat16), sdown_ref[...].astype(jnp.bfloat16),
                           preferred_element_type=jnp.float32)


def _shared_call(x, sgu, sdown):
    return pl.pallas_call(
        _shared_body,
        grid=(T // BD,),
        in_specs=[
            pl.BlockSpec((BD, D), lambda i: (i, 0)),
            pl.BlockSpec((D, 2 * SF), lambda i: (0, 0)),
            pl.BlockSpec((SF, D), lambda i: (0, 0)),
        ],
        out_specs=pl.BlockSpec((BD, D), lambda i: (i, 0)),
        out_shape=jax.ShapeDtypeStruct((T, D), jnp.float32),
    )(x, sgu, sdown)


def kernel(hidden_states, gate_w, e_score_correction_bias, expert_gate_up,
           expert_down, shared_gate_up, shared_down):
    x = hidden_states
    bias2 = e_score_correction_bias.reshape(1, E)
    slot1, slot2, w1x, w2x, te, tmb, valid = _routing_call(x, gate_w, bias2)
    s1 = slot1.reshape(T)
    s2 = slot2.reshape(T)
    te = te.reshape(NJ)
    tmb = tmb.reshape(NJ)
    valid = valid.reshape(NJ)

    xs, cwt = _dispatch(x, s1, s2, w1x, w2x)
    ys = _ffn_call(te, tmb, valid, xs, expert_gate_up, expert_down, cwt)
    shared = _shared_call(x, shared_gate_up, shared_down)
    return _collect(ys, s1, s2, shared)
